# baseline (speedup 1.0000x reference)
import functools

import jax
import jax.numpy as jnp
from jax import lax
from jax.experimental import pallas as pl
from jax.experimental.pallas import tpu as pltpu
from jax.experimental.pallas import tpu_sc as plsc


def _build_emb(B, V, D, NC, NS):
    NW = NC * NS
    b_per_w = B // NW
    CHUNK = 128
    n_chunks = b_per_w // CHUNK

    mesh = plsc.VectorSubcoreMesh(core_axis_name="c", subcore_axis_name="s")

    @functools.partial(
        pl.kernel,
        mesh=mesh,
        out_type=jax.ShapeDtypeStruct((B, D), jnp.float32),
        scratch_types=[
            pltpu.VMEM((b_per_w,), jnp.int32),
            pltpu.VMEM((CHUNK, D), jnp.float32),
            pltpu.SemaphoreType.DMA,
        ],
        compiler_params=pltpu.CompilerParams(use_tc_tiling_on_sc=False),
    )
    def emb(idx_hbm, table_hbm, out_hbm, idx_v, rows_v, sem):
        wid = lax.axis_index("s") * NC + lax.axis_index("c")
        base = wid * b_per_w
        pltpu.sync_copy(idx_hbm.at[pl.ds(base, b_per_w)], idx_v)

        def body(c, carry):
            pltpu.async_copy(
                table_hbm.at[idx_v.at[pl.ds(c * CHUNK, CHUNK)]], rows_v, sem
            ).wait()
            pltpu.sync_copy(rows_v, out_hbm.at[pl.ds(base + c * CHUNK, CHUNK)])
            return carry

        lax.fori_loop(0, n_chunks, body, 0)

    return emb


def kernel(x, table):
    Bt, H = x.shape
    V, D = table.shape
    B = Bt * H
    info = plsc.get_sparse_core_info()
    emb = _build_emb(B, V, D, info.num_cores, info.num_subcores)
    out = emb(x.reshape(B), table)
    return out.reshape(Bt, H, D)


# speedup vs baseline: 1.1167x; 1.1167x over previous
import functools

import jax
import jax.numpy as jnp
from jax import lax
from jax.experimental import pallas as pl
from jax.experimental.pallas import tpu as pltpu
from jax.experimental.pallas import tpu_sc as plsc


def _build_emb(B, V, D, NC, NS):
    NW = NC * NS
    b_per_w = B // NW
    CHUNK = 128
    n_chunks = b_per_w // CHUNK
    SLOTS = 8
    n_rounds = n_chunks // SLOTS
    assert n_chunks % SLOTS == 0

    mesh = plsc.VectorSubcoreMesh(core_axis_name="c", subcore_axis_name="s")

    @functools.partial(
        pl.kernel,
        mesh=mesh,
        out_type=jax.ShapeDtypeStruct((B, D), jnp.float32),
        scratch_types=[
            pltpu.VMEM((b_per_w,), jnp.int32),
            pltpu.VMEM((SLOTS, CHUNK, D), jnp.float32),
        ]
        + [pltpu.SemaphoreType.DMA] * (2 * SLOTS),
        compiler_params=pltpu.CompilerParams(use_tc_tiling_on_sc=False),
    )
    def emb(idx_hbm, table_hbm, out_hbm, idx_v, rows_v, *sems):
        gsem = sems[:SLOTS]
        wsem = sems[SLOTS:]
        wid = lax.axis_index("s") * NC + lax.axis_index("c")
        base = wid * b_per_w
        pltpu.sync_copy(idx_hbm.at[pl.ds(base, b_per_w)], idx_v)

        def gdesc(c, j):
            return pltpu.make_async_copy(
                table_hbm.at[idx_v.at[pl.ds(c * CHUNK, CHUNK)]],
                rows_v.at[j],
                gsem[j],
            )

        def wdesc(c, j):
            return pltpu.make_async_copy(
                rows_v.at[j],
                out_hbm.at[pl.ds(base + c * CHUNK, CHUNK)],
                wsem[j],
            )

        for j in range(SLOTS):
            gdesc(j, j).start()

        def body(t, carry):
            cb = t * SLOTS
            for j in range(SLOTS):
                gdesc(cb + j, j).wait()
                wdesc(cb + j, j).start()
            for j in range(SLOTS):
                c = cb + j
                wdesc(c, j).wait()

                @pl.when(c + SLOTS < n_chunks)
                def _():
                    gdesc(c + SLOTS, j).start()

            return carry

        lax.fori_loop(0, n_rounds, body, 0)

    return emb


def kernel(x, table):
    Bt, H = x.shape
    V, D = table.shape
    B = Bt * H
    info = plsc.get_sparse_core_info()
    emb = _build_emb(B, V, D, info.num_cores, info.num_subcores)
    out = emb(x.reshape(B), table)
    return out.reshape(Bt, H, D)


# speedup vs baseline: 1.4805x; 1.3258x over previous
import functools

import jax
import jax.numpy as jnp
from jax import lax
from jax.experimental import pallas as pl
from jax.experimental.pallas import tpu as pltpu
from jax.experimental.pallas import tpu_sc as plsc

_DPAD = 128


def _build_emb(B, V, D, NC, NS):
    NW = NC * NS
    b_per_w = B // NW
    CHUNK = 128
    n_chunks = b_per_w // CHUNK
    SLOTS = 4
    n_rounds = n_chunks // SLOTS
    assert n_chunks % SLOTS == 0

    mesh = plsc.VectorSubcoreMesh(core_axis_name="c", subcore_axis_name="s")

    @functools.partial(
        pl.kernel,
        mesh=mesh,
        out_type=jax.ShapeDtypeStruct((B, _DPAD), jnp.float32),
        scratch_types=[
            pltpu.VMEM((b_per_w,), jnp.int32),
            pltpu.VMEM((SLOTS, CHUNK, D), jnp.float32),
        ]
        + [pltpu.SemaphoreType.DMA] * (2 * SLOTS),
        compiler_params=pltpu.CompilerParams(use_tc_tiling_on_sc=False),
    )
    def emb(idx_hbm, table_hbm, out_hbm, idx_v, rows_v, *sems):
        gsem = sems[:SLOTS]
        wsem = sems[SLOTS:]
        wid = lax.axis_index("s") * NC + lax.axis_index("c")
        base = wid * b_per_w
        pltpu.sync_copy(idx_hbm.at[pl.ds(base, b_per_w)], idx_v)

        def gdesc(c, j):
            return pltpu.make_async_copy(
                table_hbm.at[idx_v.at[pl.ds(c * CHUNK, CHUNK)]],
                rows_v.at[j],
                gsem[j],
            )

        def wdesc(c, j):
            return pltpu.make_async_copy(
                rows_v.at[j],
                out_hbm.at[pl.ds(base + c * CHUNK, CHUNK), pl.ds(0, D)],
                wsem[j],
            )

        for j in range(SLOTS):
            gdesc(j, j).start()

        def body(t, carry):
            cb = t * SLOTS
            for j in range(SLOTS):
                gdesc(cb + j, j).wait()
                wdesc(cb + j, j).start()
            for j in range(SLOTS):
                c = cb + j
                wdesc(c, j).wait()

                @pl.when(c + SLOTS < n_chunks)
                def _():
                    gdesc(c + SLOTS, j).start()

            return carry

        lax.fori_loop(0, n_rounds, body, 0)

    return emb


def kernel(x, table):
    Bt, H = x.shape
    V, D = table.shape
    B = Bt * H
    info = plsc.get_sparse_core_info()
    emb = _build_emb(B, V, D, info.num_cores, info.num_subcores)
    out = emb(x.reshape(B), table)
    return out[:, :D].reshape(Bt, H, D)
